# Initial kernel scaffold; baseline (speedup 1.0000x reference)
import jax
import jax.numpy as jnp
from jax.experimental import pallas as pl


def kernel(x, expert_weights, expert_indices, w1, w2, w3):
    raise NotImplementedError("write your pallas kernel here")


# speedup vs baseline: 4.7838x; 4.7838x over previous
import jax
import jax.numpy as jnp
from jax.experimental import pallas as pl

T = 8
DIM = 1024
INTER = 2816
E = 8
TOPK = 2

BI = 1408
NJ = INTER // BI


def _ffn_kernel(ew_ref, idx_ref, x_ref, w1_ref, w3_ref, w2_ref, out_ref):
    e = pl.program_id(0)
    j = pl.program_id(1)

    @pl.when(jnp.logical_and(e == 0, j == 0))
    def _init():
        out_ref[...] = jnp.zeros_like(out_ref)

    ew = ew_ref[...]
    ewn = ew / jnp.sum(ew, axis=-1, keepdims=True)
    idx = idx_ref[...]
    scale = jnp.sum(jnp.where(idx == e, ewn, 0.0), axis=-1, keepdims=True)

    x = x_ref[...]
    w1 = w1_ref[0]
    w3 = w3_ref[0]
    w2 = w2_ref[0]

    h1 = jax.lax.dot_general(x, w1, (((1,), (1,)), ((), ())),
                             preferred_element_type=jnp.float32)
    h3 = jax.lax.dot_general(x, w3, (((1,), (1,)), ((), ())),
                             preferred_element_type=jnp.float32)
    h = (h1 * jax.nn.sigmoid(h1)) * h3 * scale

    contrib = jax.lax.dot_general(h, w2, (((1,), (1,)), ((), ())),
                                  preferred_element_type=jnp.float32)
    out_ref[...] += contrib


def kernel(x, expert_weights, expert_indices, w1, w2, w3):
    idx = expert_indices.astype(jnp.int32)
    grid = (E, NJ)
    return pl.pallas_call(
        _ffn_kernel,
        grid=grid,
        in_specs=[
            pl.BlockSpec((T, TOPK), lambda e, j: (0, 0)),
            pl.BlockSpec((T, TOPK), lambda e, j: (0, 0)),
            pl.BlockSpec((T, DIM), lambda e, j: (0, 0)),
            pl.BlockSpec((1, BI, DIM), lambda e, j: (e, j, 0)),
            pl.BlockSpec((1, BI, DIM), lambda e, j: (e, j, 0)),
            pl.BlockSpec((1, DIM, BI), lambda e, j: (e, 0, j)),
        ],
        out_specs=pl.BlockSpec((T, DIM), lambda e, j: (0, 0)),
        out_shape=jax.ShapeDtypeStruct((T, DIM), jnp.float32),
    )(expert_weights, idx, x, w1, w3, w2)
